# Initial kernel scaffold; baseline (speedup 1.0000x reference)
import jax
import jax.numpy as jnp
from jax.experimental import pallas as pl


def kernel(inputs):
    raise NotImplementedError("write your pallas kernel here")


# speedup vs baseline: 3.4005x; 3.4005x over previous
import functools

import jax
import jax.numpy as jnp
from jax import lax
from jax.experimental import pallas as pl
from jax.experimental.pallas import tpu as pltpu
from jax.experimental.pallas import tpu_sc as plsc

_K = 8
_L = 16
_NC, _NS = 2, 16
_NW = _NC * _NS
_B, _N = 128, 32768
_RPW = _B // _NW
_CHUNKS = _N // _L
_GROUP = 32
_NG = _CHUNKS // _GROUP
_CAP = 256
_CHK = _CAP // _L
_NEG = -3.0e38
_BIGI = 1 << 30

_mesh = plsc.VectorSubcoreMesh(
    core_axis_name="c", subcore_axis_name="s",
    num_cores=_NC, num_subcores=_NS)


@functools.partial(
    pl.kernel,
    out_type=jax.ShapeDtypeStruct((_B, _N), jnp.float32),
    mesh=_mesh,
    compiler_params=pltpu.CompilerParams(needs_layout_passes=False),
    scratch_types=[
        pltpu.VMEM((_N,), jnp.float32),
        pltpu.VMEM((_N,), jnp.float32),
        pltpu.VMEM((_NG * _L,), jnp.float32),
        pltpu.VMEM((_CAP,), jnp.float32),
        pltpu.VMEM((_CAP,), jnp.int32),
    ],
)
def _topk_softmax(in_hbm, out_hbm, row_v, zero_v, gm_v, cand_v, cand_i):
    wid = lax.axis_index("s") * _NC + lax.axis_index("c")
    lane = lax.iota(jnp.int32, _L)
    zeros16 = jnp.zeros((_L,), jnp.float32)

    def z_body(i, _):
        base = i * (_L * 16)
        for u in range(16):
            zero_v[pl.ds(base + u * _L, _L)] = zeros16
        return 0
    lax.fori_loop(0, _CHUNKS // 16, z_body, jnp.int32(0))

    def row_body(r, _):
        row = wid * _RPW + r

        pltpu.sync_copy(in_hbm.at[row], row_v)

        def g_body(g, rmax):
            base = g * (_GROUP * _L)
            gm = row_v[pl.ds(base, _L)]
            for c in range(1, _GROUP):
                gm = jnp.maximum(gm, row_v[pl.ds(base + c * _L, _L)])
            gm_v[pl.ds(g * _L, _L)] = gm
            return jnp.maximum(rmax, gm)
        rmax = lax.fori_loop(0, _NG, g_body, jnp.full((_L,), _NEG, jnp.float32))

        d, _unused = plsc.sort_key_val(rmax, rmax, descending=True)
        t = jnp.min(jnp.where(lane < _K, d, 3.0e38))

        for j in range(_CHK):
            cand_v[pl.ds(j * _L, _L)] = jnp.full((_L,), _NEG, jnp.float32)
            cand_i[pl.ds(j * _L, _L)] = jnp.full((_L,), _BIGI, jnp.int32)

        def coll_body(g, ptr):
            gm = gm_v[pl.ds(g * _L, _L)]
            hit = jnp.any(gm >= t)

            def do(p):
                base = g * (_GROUP * _L)
                for c in range(_GROUP):
                    x = row_v[pl.ds(base + c * _L, _L)]
                    m = x >= t
                    mi = m.astype(jnp.int32)
                    pos = jnp.clip(p + plsc.cumsum(mi) - 1, 0, _CAP - 1)
                    col = base + c * _L + lane
                    plsc.store_scatter(cand_v, [pos], x, mask=m)
                    plsc.store_scatter(cand_i, [pos], col, mask=m)
                    p = jnp.minimum(p + jnp.sum(mi), _CAP)
                return p
            return lax.cond(hit, do, lambda p: p, ptr)
        ptr = lax.fori_loop(0, _NG, coll_body, jnp.int32(0))

        nv = jnp.minimum((ptr + _L - 1) // _L, _CHK)
        tv = jnp.full((_L,), _NEG, jnp.float32)
        ti = jnp.full((_L,), _BIGI, jnp.int32)
        for k in range(_K):
            def sel_body(j, carry):
                bv, bi = carry
                v = cand_v[pl.ds(j * _L, _L)]
                i = cand_i[pl.ds(j * _L, _L)]
                better = (v > bv) | ((v == bv) & (i < bi))
                return jnp.where(better, v, bv), jnp.where(better, i, bi)
            bv, bi = lax.fori_loop(
                0, nv, sel_body,
                (jnp.full((_L,), _NEG, jnp.float32),
                 jnp.full((_L,), _BIGI, jnp.int32)))
            vstar = jnp.max(bv)
            istar = jnp.min(jnp.where(bv == vstar, bi, _BIGI))
            tv = jnp.where(lane == k, vstar, tv)
            ti = jnp.where(lane == k, istar, ti)

            def rem_body(j, _):
                v = cand_v[pl.ds(j * _L, _L)]
                i = cand_i[pl.ds(j * _L, _L)]
                cand_v[pl.ds(j * _L, _L)] = jnp.where(i == istar, _NEG, v)
                return 0
            lax.fori_loop(0, nv, rem_body, jnp.int32(0))

        mx = jnp.max(tv)
        e = jnp.exp(tv - mx)
        gates = e / jnp.sum(e)

        kmask = lane < _K
        plsc.store_scatter(zero_v, [ti], gates, mask=kmask)
        pltpu.sync_copy(zero_v, out_hbm.at[row])
        plsc.store_scatter(zero_v, [ti], zeros16, mask=kmask)
        return 0

    lax.fori_loop(0, _RPW, row_body, jnp.int32(0))


def kernel(inputs):
    return _topk_softmax(inputs)


# speedup vs baseline: 3.5569x; 1.0460x over previous
import functools

import jax
import jax.numpy as jnp
from jax import lax
from jax.experimental import pallas as pl
from jax.experimental.pallas import tpu as pltpu
from jax.experimental.pallas import tpu_sc as plsc

_K = 8
_L = 16
_NC, _NS = 2, 16
_NW = _NC * _NS
_B, _N = 128, 32768
_RPW = _B // _NW
_CHUNKS = _N // _L
_GROUP = 32
_NG = _CHUNKS // _GROUP
_CAP = 256
_CHK = _CAP // _L
_NEG = -3.0e38
_BIGI = 1 << 30

_mesh = plsc.VectorSubcoreMesh(
    core_axis_name="c", subcore_axis_name="s",
    num_cores=_NC, num_subcores=_NS)


@functools.partial(
    pl.kernel,
    out_type=jax.ShapeDtypeStruct((_B, _N), jnp.float32),
    mesh=_mesh,
    compiler_params=pltpu.CompilerParams(needs_layout_passes=False),
    scratch_types=[
        pltpu.VMEM((_N,), jnp.float32),
        pltpu.VMEM((_N,), jnp.float32),
        pltpu.VMEM((_N,), jnp.float32),
        pltpu.VMEM((_NG * _L,), jnp.float32),
        pltpu.VMEM((_CAP,), jnp.float32),
        pltpu.VMEM((_CAP,), jnp.int32),
        pltpu.SemaphoreType.DMA,
        pltpu.SemaphoreType.DMA,
        pltpu.SemaphoreType.DMA,
    ],
)
def _topk_softmax(in_hbm, out_hbm, row_a, row_b, zero_v, gm_v,
                  cand_v, cand_i, sem_a, sem_b, sem_o):
    wid = lax.axis_index("s") * _NC + lax.axis_index("c")
    lane = lax.iota(jnp.int32, _L)
    zeros16 = jnp.zeros((_L,), jnp.float32)
    kmask = lane < _K
    bufs = [row_a, row_b]
    sems = [sem_a, sem_b]
    rows = [wid * _RPW + r for r in range(_RPW)]

    in_descs = [pltpu.async_copy(in_hbm.at[rows[0]], row_a, sem_a)]

    def z_body(i, _):
        base = i * (_L * 16)
        for u in range(16):
            zero_v[pl.ds(base + u * _L, _L)] = zeros16
        return 0
    lax.fori_loop(0, _CHUNKS // 16, z_body, jnp.int32(0))

    def top8_of_row(row_v):
        def g_body(g, rmax):
            base = g * (_GROUP * _L)
            gm = row_v[pl.ds(base, _L)]
            for c in range(1, _GROUP):
                gm = jnp.maximum(gm, row_v[pl.ds(base + c * _L, _L)])
            gm_v[pl.ds(g * _L, _L)] = gm
            return jnp.maximum(rmax, gm)
        rmax = lax.fori_loop(0, _NG, g_body,
                             jnp.full((_L,), _NEG, jnp.float32))

        d, _unused = plsc.sort_key_val(rmax, rmax, descending=True)
        t = jnp.min(jnp.where(kmask, d, 3.0e38))

        for j in range(_CHK):
            cand_v[pl.ds(j * _L, _L)] = jnp.full((_L,), _NEG, jnp.float32)
            cand_i[pl.ds(j * _L, _L)] = jnp.full((_L,), _BIGI, jnp.int32)

        def coll_body(g, ptr):
            gm = gm_v[pl.ds(g * _L, _L)]
            hit = jnp.any(gm >= t)

            def do(p):
                base = g * (_GROUP * _L)
                for c in range(_GROUP):
                    x = row_v[pl.ds(base + c * _L, _L)]
                    m = x >= t
                    mi = m.astype(jnp.int32)
                    pos = jnp.clip(p + plsc.cumsum(mi) - 1, 0, _CAP - 1)
                    col = base + c * _L + lane
                    plsc.store_scatter(cand_v, [pos], x, mask=m)
                    plsc.store_scatter(cand_i, [pos], col, mask=m)
                    p = jnp.minimum(p + jnp.sum(mi), _CAP)
                return p
            return lax.cond(hit, do, lambda p: p, ptr)
        ptr = lax.fori_loop(0, _NG, coll_body, jnp.int32(0))

        nv = jnp.minimum((ptr + _L - 1) // _L, _CHK)
        tv = jnp.full((_L,), _NEG, jnp.float32)
        ti = jnp.full((_L,), _BIGI, jnp.int32)
        for k in range(_K):
            def sel_body(j, carry):
                bv, bi = carry
                v = cand_v[pl.ds(j * _L, _L)]
                i = cand_i[pl.ds(j * _L, _L)]
                better = (v > bv) | ((v == bv) & (i < bi))
                return jnp.where(better, v, bv), jnp.where(better, i, bi)
            bv, bi = lax.fori_loop(
                0, nv, sel_body,
                (jnp.full((_L,), _NEG, jnp.float32),
                 jnp.full((_L,), _BIGI, jnp.int32)))
            vstar = jnp.max(bv)
            istar = jnp.min(jnp.where(bv == vstar, bi, _BIGI))
            tv = jnp.where(lane == k, vstar, tv)
            ti = jnp.where(lane == k, istar, ti)

            def rem_body(j, _):
                v = cand_v[pl.ds(j * _L, _L)]
                i = cand_i[pl.ds(j * _L, _L)]
                cand_v[pl.ds(j * _L, _L)] = jnp.where(i == istar, _NEG, v)
                return 0
            lax.fori_loop(0, nv, rem_body, jnp.int32(0))

        mx = jnp.max(tv)
        e = jnp.exp(tv - mx)
        return ti, e / jnp.sum(e)

    out_desc = None
    prev_ti = None
    for r in range(_RPW):
        in_descs[r].wait()
        if r + 1 < _RPW:
            in_descs.append(pltpu.async_copy(
                in_hbm.at[rows[r + 1]], bufs[(r + 1) % 2],
                sems[(r + 1) % 2]))
        ti, gates = top8_of_row(bufs[r % 2])
        if out_desc is not None:
            out_desc.wait()
            plsc.store_scatter(zero_v, [prev_ti], zeros16, mask=kmask)
        plsc.store_scatter(zero_v, [ti], gates, mask=kmask)
        out_desc = pltpu.async_copy(zero_v, out_hbm.at[rows[r]], sem_o)
        prev_ti = ti
    out_desc.wait()


def kernel(inputs):
    return _topk_softmax(inputs)


# speedup vs baseline: 4.2078x; 1.1830x over previous
import functools

import jax
import jax.numpy as jnp
from jax import lax
from jax.experimental import pallas as pl
from jax.experimental.pallas import tpu as pltpu
from jax.experimental.pallas import tpu_sc as plsc

_K = 8
_L = 16
_NC, _NS = 2, 16
_NW = _NC * _NS
_B, _N = 128, 32768
_RPW = _B // _NW
_CHUNKS = _N // _L
_GROUP = 32
_NG = _CHUNKS // _GROUP
_SLOTS = 32
_CAP = _SLOTS * _L
_CHK = _CAP // _L
_NEG = -3.0e38
_BIGI = 1 << 30

_mesh = plsc.VectorSubcoreMesh(
    core_axis_name="c", subcore_axis_name="s",
    num_cores=_NC, num_subcores=_NS)


@functools.partial(
    pl.kernel,
    out_type=jax.ShapeDtypeStruct((_B, _N), jnp.float32),
    mesh=_mesh,
    compiler_params=pltpu.CompilerParams(needs_layout_passes=False),
    scratch_types=[
        pltpu.VMEM((_N,), jnp.float32),
        pltpu.VMEM((_N,), jnp.float32),
        pltpu.VMEM((_N,), jnp.float32),
        pltpu.VMEM((_NG * _L,), jnp.float32),
        pltpu.VMEM((_CAP,), jnp.float32),
        pltpu.VMEM((_CAP,), jnp.int32),
        pltpu.SemaphoreType.DMA,
        pltpu.SemaphoreType.DMA,
        pltpu.SemaphoreType.DMA,
    ],
)
def _topk_softmax(in_hbm, out_hbm, row_a, row_b, zero_v, gm_v,
                  cand_v, cand_i, sem_a, sem_b, sem_o):
    wid = lax.axis_index("s") * _NC + lax.axis_index("c")
    lane = lax.iota(jnp.int32, _L)
    zeros16 = jnp.zeros((_L,), jnp.float32)
    kmask = lane < _K
    bufs = [row_a, row_b]
    sems = [sem_a, sem_b]
    rows = [wid * _RPW + r for r in range(_RPW)]

    in_descs = [pltpu.async_copy(in_hbm.at[rows[0]], row_a, sem_a)]

    def z_body(i, _):
        base = i * (_L * 16)
        for u in range(16):
            zero_v[pl.ds(base + u * _L, _L)] = zeros16
        return 0
    lax.fori_loop(0, _CHUNKS // 16, z_body, jnp.int32(0))

    def top8_of_row(row_v):
        def g_body(g, rmax):
            base = g * (_GROUP * _L)
            acc = [row_v[pl.ds(base + c * _L, _L)] for c in range(4)]
            for c in range(4, _GROUP, 4):
                for u in range(4):
                    acc[u] = jnp.maximum(
                        acc[u], row_v[pl.ds(base + (c + u) * _L, _L)])
            gm = jnp.maximum(jnp.maximum(acc[0], acc[1]),
                             jnp.maximum(acc[2], acc[3]))
            gm_v[pl.ds(g * _L, _L)] = gm
            return jnp.maximum(rmax, gm)
        rmax = lax.fori_loop(0, _NG, g_body,
                             jnp.full((_L,), _NEG, jnp.float32))

        d, _unused = plsc.sort_key_val(rmax, rmax, descending=True)
        t = jnp.min(jnp.where(kmask, d, 3.0e38))

        for j in range(_CHK):
            cand_v[pl.ds(j * _L, _L)] = jnp.full((_L,), _NEG, jnp.float32)
            cand_i[pl.ds(j * _L, _L)] = jnp.full((_L,), _BIGI, jnp.int32)

        def coll_body(g, cnt):
            gm = gm_v[pl.ds(g * _L, _L)]
            hit = jnp.any(gm >= t)

            def do(c0):
                base = g * (_GROUP * _L)
                for c in range(_GROUP):
                    x = row_v[pl.ds(base + c * _L, _L)]
                    m = x >= t
                    pos = jnp.minimum(c0, _SLOTS - 1) * _L + lane
                    col = base + c * _L + lane
                    plsc.store_scatter(cand_v, [pos], x, mask=m)
                    plsc.store_scatter(cand_i, [pos], col, mask=m)
                    c0 = c0 + m.astype(jnp.int32)
                return c0
            return lax.cond(hit, do, lambda c0: c0, cnt)
        cnt = lax.fori_loop(0, _NG, coll_body,
                            jnp.zeros((_L,), jnp.int32))

        nv = jnp.minimum(jnp.max(cnt), _CHK)
        tv = jnp.full((_L,), _NEG, jnp.float32)
        ti = jnp.full((_L,), _BIGI, jnp.int32)
        for k in range(_K):
            def sel_body(j, carry):
                bv, bi = carry
                v = cand_v[pl.ds(j * _L, _L)]
                i = cand_i[pl.ds(j * _L, _L)]
                better = (v > bv) | ((v == bv) & (i < bi))
                return jnp.where(better, v, bv), jnp.where(better, i, bi)
            bv, bi = lax.fori_loop(
                0, nv, sel_body,
                (jnp.full((_L,), _NEG, jnp.float32),
                 jnp.full((_L,), _BIGI, jnp.int32)))
            vstar = jnp.max(bv)
            istar = jnp.min(jnp.where(bv == vstar, bi, _BIGI))
            tv = jnp.where(lane == k, vstar, tv)
            ti = jnp.where(lane == k, istar, ti)

            def rem_body(j, _):
                v = cand_v[pl.ds(j * _L, _L)]
                i = cand_i[pl.ds(j * _L, _L)]
                cand_v[pl.ds(j * _L, _L)] = jnp.where(i == istar, _NEG, v)
                return 0
            lax.fori_loop(0, nv, rem_body, jnp.int32(0))

        mx = jnp.max(tv)
        e = jnp.exp(tv - mx)
        return ti, e / jnp.sum(e)

    out_desc = None
    prev_ti = None
    for r in range(_RPW):
        in_descs[r].wait()
        if r + 1 < _RPW:
            in_descs.append(pltpu.async_copy(
                in_hbm.at[rows[r + 1]], bufs[(r + 1) % 2],
                sems[(r + 1) % 2]))
        ti, gates = top8_of_row(bufs[r % 2])
        if out_desc is not None:
            out_desc.wait()
            plsc.store_scatter(zero_v, [prev_ti], zeros16, mask=kmask)
        plsc.store_scatter(zero_v, [ti], gates, mask=kmask)
        out_desc = pltpu.async_copy(zero_v, out_hbm.at[rows[r]], sem_o)
        prev_ti = ti
    out_desc.wait()


def kernel(inputs):
    return _topk_softmax(inputs)
